# baseline (speedup 1.0000x reference)
import functools

import jax
import jax.numpy as jnp
from jax import lax
from jax.experimental import pallas as pl
from jax.experimental.pallas import tpu as pltpu
from jax.experimental.pallas import tpu_sc as plsc

N = 10000
E = 320000
D = 128
EPS_BN = 1e-5
EPS_LN = 1e-5

NC = 2
NS = 16
NW = NC * NS
EPT = E // NW
EPT_PAD = 10240
CH = 128
NCHUNK = EPT_PAD // CH
CPI = 8
NIT = NCHUNK // CPI
ACC_N = 10112
RPT = ACC_N // NS


@functools.cache
def _make_seg_sum():
  mesh = plsc.VectorSubcoreMesh(core_axis_name="c", subcore_axis_name="s",
                                num_cores=NC, num_subcores=NS)

  @functools.partial(
      pl.kernel,
      out_type=jax.ShapeDtypeStruct((NC, ACC_N, D), jnp.float32),
      mesh=mesh,
      scratch_types=[
          pltpu.VMEM_SHARED((ACC_N, D), jnp.float32),
          pltpu.VMEM((CPI, 2, CH), jnp.int32),
          pltpu.VMEM((CPI, 2, CH), jnp.int32),
          pltpu.VMEM((CH, D), jnp.float32),
          pltpu.SemaphoreType.DMA,
          pltpu.SemaphoreType.DMA,
          pltpu.SemaphoreType.DMA,
      ],
  )
  def seg(h_hbm, idx_hbm, z_hbm, out_hbm,
          acc, ib0, ib1, rows, gsem, is0, is1):
    cid = lax.axis_index("c")
    sid = lax.axis_index("s")
    pltpu.sync_copy(z_hbm, acc.at[pl.ds(sid * RPT, RPT)])
    plsc.subcore_barrier()

    ibufs = [ib0, ib1]
    isems = [is0, is1]

    def iload(t, p):
      pltpu.async_copy(idx_hbm.at[cid, sid, pl.ds(t * CPI, CPI)],
                       ibufs[p], isems[p])

    def iwait(p):
      pltpu.make_async_copy(idx_hbm.at[cid, sid, pl.ds(0, CPI)],
                            ibufs[p], isems[p]).wait()

    iload(0, 0)

    def one_iter(t, p):
      iwait(p)
      iload(jnp.minimum(t + 1, NIT - 1), 1 - p)
      for k in range(CPI):
        pltpu.async_copy(h_hbm.at[ibufs[p].at[k, 0]], rows, gsem).wait()
        pltpu.sync_copy(rows, acc.at[ibufs[p].at[k, 1]], add=True)

    def body(s, carry):
      one_iter(2 * s, 0)
      one_iter(2 * s + 1, 1)
      return carry

    lax.fori_loop(0, NIT // 2, body, 0)
    iwait(0)
    plsc.subcore_barrier()
    pltpu.sync_copy(acc.at[pl.ds(sid * RPT, RPT)],
                    out_hbm.at[cid, pl.ds(sid * RPT, RPT)])

  return seg


def _seg_sum(h, idx, zeros):
  return _make_seg_sum()(h, idx, zeros)


BR = 1000
NB = N // BR


def _tc_body_last(h_ref, p_ref, W1_ref, b1_ref, g1_ref, be1_ref,
                  W2_ref, b2_ref, g2_ref, be2_ref,
                  out_ref, mean_ref, max_ref, add_ref,
                  y_sc, s1, s2, psum, pmax):
  _tc_common(h_ref, p_ref, W1_ref, b1_ref, g1_ref, be1_ref,
             W2_ref, b2_ref, g2_ref, be2_ref, out_ref,
             y_sc, s1, s2,
             pool=(mean_ref, max_ref, add_ref, psum, pmax))


def _tc_body_mid(h_ref, p_ref, W1_ref, b1_ref, g1_ref, be1_ref,
                 W2_ref, b2_ref, g2_ref, be2_ref,
                 out_ref, y_sc, s1, s2):
  _tc_common(h_ref, p_ref, W1_ref, b1_ref, g1_ref, be1_ref,
             W2_ref, b2_ref, g2_ref, be2_ref, out_ref,
             y_sc, s1, s2, pool=None)


def _tc_common(h_ref, p_ref, W1_ref, b1_ref, g1_ref, be1_ref,
               W2_ref, b2_ref, g2_ref, be2_ref, out_ref,
               y_sc, s1, s2, pool):
  i = pl.program_id(0)
  j = pl.program_id(1)

  @pl.when(i == 0)
  def _phase0():
    m = h_ref[...] + p_ref[0] + p_ref[1]
    y = jnp.dot(m, W1_ref[...], preferred_element_type=jnp.float32)
    y = y + b1_ref[...]
    y_sc[pl.ds(j * BR, BR), :] = y

    @pl.when(j == 0)
    def _():
      s1[...] = jnp.zeros((1, D), jnp.float32)
      s2[...] = jnp.zeros((1, D), jnp.float32)

    s1[...] += jnp.sum(y, axis=0, keepdims=True)
    s2[...] += jnp.sum(y * y, axis=0, keepdims=True)

  @pl.when(i == 1)
  def _phase1():
    mu = s1[...] / N
    var = s2[...] / N - mu * mu
    y = y_sc[pl.ds(j * BR, BR), :]
    yn = (y - mu) * lax.rsqrt(var + EPS_BN) * g1_ref[...] + be1_ref[...]
    yn = jnp.maximum(yn, 0.0)
    z = jnp.dot(yn, W2_ref[...], preferred_element_type=jnp.float32)
    z = z + b2_ref[...]
    mu2 = jnp.mean(z, axis=1, keepdims=True)
    var2 = jnp.mean(z * z, axis=1, keepdims=True) - mu2 * mu2
    z = (z - mu2) * lax.rsqrt(var2 + EPS_LN) * g2_ref[...] + be2_ref[...]
    o = z + h_ref[...]
    out_ref[...] = o

    if pool is not None:
      mean_ref, max_ref, add_ref, psum, pmax = pool

      @pl.when(j == 0)
      def _():
        psum[...] = jnp.zeros((1, D), jnp.float32)
        pmax[...] = jnp.full((1, D), -jnp.inf, jnp.float32)

      psum[...] += jnp.sum(o, axis=0, keepdims=True)
      pmax[...] = jnp.maximum(pmax[...], jnp.max(o, axis=0, keepdims=True))

      @pl.when(j == NB - 1)
      def _():
        s = psum[...]
        add_ref[...] = s
        mean_ref[...] = s / N
        max_ref[...] = pmax[...]


def _tc_layer(h, parts, W1, b1, g1, be1, W2, b2, g2, be2, last):
  row_spec = pl.BlockSpec((BR, D), lambda i, j: (j, 0))
  p_spec = pl.BlockSpec((NC, BR, D), lambda i, j: (0, (1 - i) * j, 0))
  w_spec = pl.BlockSpec((D, D), lambda i, j: (0, 0))
  v_spec = pl.BlockSpec((1, D), lambda i, j: (0, 0))
  out_row_spec = pl.BlockSpec((BR, D), lambda i, j: (i * j, 0))

  in_specs = [row_spec, p_spec, w_spec, v_spec, v_spec, v_spec,
              w_spec, v_spec, v_spec, v_spec]
  scratch = [
      pltpu.VMEM((N, D), jnp.float32),
      pltpu.VMEM((1, D), jnp.float32),
      pltpu.VMEM((1, D), jnp.float32),
  ]
  if last:
    out_shape = [
        jax.ShapeDtypeStruct((N, D), jnp.float32),
        jax.ShapeDtypeStruct((1, D), jnp.float32),
        jax.ShapeDtypeStruct((1, D), jnp.float32),
        jax.ShapeDtypeStruct((1, D), jnp.float32),
    ]
    out_specs = [out_row_spec, v_spec, v_spec, v_spec]
    body = _tc_body_last
    scratch += [pltpu.VMEM((1, D), jnp.float32),
                pltpu.VMEM((1, D), jnp.float32)]
  else:
    out_shape = jax.ShapeDtypeStruct((N, D), jnp.float32)
    out_specs = out_row_spec
    body = _tc_body_mid

  return pl.pallas_call(
      body,
      grid=(2, NB),
      in_specs=in_specs,
      out_specs=out_specs,
      out_shape=out_shape,
      scratch_shapes=scratch,
  )(h, parts, W1, b1, g1, be1, W2, b2, g2, be2)


def kernel(x, edge_index,
           W1_0, b1_0, bng_0, bnb_0, W2_0, b2_0, lng_0, lnb_0,
           W1_1, b1_1, bng_1, bnb_1, W2_1, b2_1, lng_1, lnb_1,
           W1_2, b1_2, bng_2, bnb_2, W2_2, b2_2, lng_2, lnb_2):
  pad = ((0, 0), (0, 0), (0, EPT_PAD - EPT))
  ei = jnp.pad(edge_index.reshape(2, NW, EPT), pad,
               constant_values=N - 1)
  ei = ei.at[1, :, EPT:].set(N)
  idx = jnp.stack(
      [ei[0].reshape(NW, NCHUNK, CH), ei[1].reshape(NW, NCHUNK, CH)],
      axis=2).reshape(NC, NS, NCHUNK, 2, CH)
  zeros = jnp.zeros((RPT, D), jnp.float32)
  params = [
      (W1_0, b1_0, bng_0, bnb_0, W2_0, b2_0, lng_0, lnb_0),
      (W1_1, b1_1, bng_1, bnb_1, W2_1, b2_1, lng_1, lnb_1),
      (W1_2, b1_2, bng_2, bnb_2, W2_2, b2_2, lng_2, lnb_2),
  ]
  h = x
  outs = None
  for i in range(3):
    W1, b1, g1, be1, W2, b2, g2, be2 = params[i]
    parts = _seg_sum(h, idx, zeros)
    res = _tc_layer(h, parts,
                    W1, b1.reshape(1, D), g1.reshape(1, D), be1.reshape(1, D),
                    W2, b2.reshape(1, D), g2.reshape(1, D), be2.reshape(1, D),
                    last=(i == 2))
    if i == 2:
      h, mean_p, max_p, add_p = res
      outs = (mean_p, max_p, add_p, h)
    else:
      h = res
  return outs


# speedup vs baseline: 2.1588x; 2.1588x over previous
import functools

import jax
import jax.numpy as jnp
from jax import lax
from jax.experimental import pallas as pl
from jax.experimental.pallas import tpu as pltpu
from jax.experimental.pallas import tpu_sc as plsc

N = 10000
E = 320000
D = 128
EPS_BN = 1e-5
EPS_LN = 1e-5

NC = 2
NS = 16
NW = NC * NS
EPT = E // NW
CH = 100
NCHUNK = EPT // CH
CPI = 10
NIT = NCHUNK // CPI
ACC_N = 10112
RPT = ACC_N // NS


@functools.cache
def _make_seg_sum():
  mesh = plsc.VectorSubcoreMesh(core_axis_name="c", subcore_axis_name="s",
                                num_cores=NC, num_subcores=NS)

  @functools.partial(
      pl.kernel,
      out_type=jax.ShapeDtypeStruct((NC, ACC_N, D), jnp.float32),
      mesh=mesh,
      scratch_types=[
          pltpu.VMEM_SHARED((ACC_N, D), jnp.float32),
          pltpu.VMEM((CPI, 2, CH), jnp.int32),
          pltpu.VMEM((CPI, 2, CH), jnp.int32),
          pltpu.VMEM((CH, D), jnp.float32),
          pltpu.SemaphoreType.DMA,
          pltpu.SemaphoreType.DMA,
          pltpu.SemaphoreType.DMA,
      ],
  )
  def seg(h_hbm, idx_hbm, z_hbm, out_hbm,
          acc, ib0, ib1, rows, gsem, is0, is1):
    cid = lax.axis_index("c")
    sid = lax.axis_index("s")
    pltpu.sync_copy(z_hbm, acc.at[pl.ds(sid * RPT, RPT)])
    plsc.subcore_barrier()

    ibufs = [ib0, ib1]
    isems = [is0, is1]

    def iload(t, p):
      pltpu.async_copy(idx_hbm.at[cid, sid, pl.ds(t * CPI, CPI)],
                       ibufs[p], isems[p])

    def iwait(p):
      pltpu.make_async_copy(idx_hbm.at[cid, sid, pl.ds(0, CPI)],
                            ibufs[p], isems[p]).wait()

    iload(0, 0)

    def one_iter(t, p):
      iwait(p)
      iload(jnp.minimum(t + 1, NIT - 1), 1 - p)
      for k in range(CPI):
        pltpu.async_copy(h_hbm.at[ibufs[p].at[k, 0]], rows, gsem).wait()
        pltpu.sync_copy(rows, acc.at[ibufs[p].at[k, 1]], add=True)

    def body(s, carry):
      one_iter(2 * s, 0)
      one_iter(2 * s + 1, 1)
      return carry

    lax.fori_loop(0, NIT // 2, body, 0)
    iwait(0)
    plsc.subcore_barrier()
    pltpu.sync_copy(acc.at[pl.ds(sid * RPT, RPT)],
                    out_hbm.at[cid, pl.ds(sid * RPT, RPT)])

  return seg


def _seg_sum(h, idx, zeros):
  return _make_seg_sum()(h, idx, zeros)


BR = 1000
NB = N // BR


def _tc_body_last(h_ref, p_ref, W1_ref, b1_ref, g1_ref, be1_ref,
                  W2_ref, b2_ref, g2_ref, be2_ref,
                  out_ref, mean_ref, max_ref, add_ref,
                  y_sc, s1, s2, psum, pmax):
  _tc_common(h_ref, p_ref, W1_ref, b1_ref, g1_ref, be1_ref,
             W2_ref, b2_ref, g2_ref, be2_ref, out_ref,
             y_sc, s1, s2,
             pool=(mean_ref, max_ref, add_ref, psum, pmax))


def _tc_body_mid(h_ref, p_ref, W1_ref, b1_ref, g1_ref, be1_ref,
                 W2_ref, b2_ref, g2_ref, be2_ref,
                 out_ref, y_sc, s1, s2):
  _tc_common(h_ref, p_ref, W1_ref, b1_ref, g1_ref, be1_ref,
             W2_ref, b2_ref, g2_ref, be2_ref, out_ref,
             y_sc, s1, s2, pool=None)


def _tc_common(h_ref, p_ref, W1_ref, b1_ref, g1_ref, be1_ref,
               W2_ref, b2_ref, g2_ref, be2_ref, out_ref,
               y_sc, s1, s2, pool):
  i = pl.program_id(0)
  j = pl.program_id(1)

  @pl.when(i == 0)
  def _phase0():
    m = h_ref[...] + p_ref[0] + p_ref[1]
    y = jnp.dot(m, W1_ref[...], preferred_element_type=jnp.float32)
    y = y + b1_ref[...]
    y_sc[pl.ds(j * BR, BR), :] = y

    @pl.when(j == 0)
    def _():
      s1[...] = jnp.zeros((1, D), jnp.float32)
      s2[...] = jnp.zeros((1, D), jnp.float32)

    s1[...] += jnp.sum(y, axis=0, keepdims=True)
    s2[...] += jnp.sum(y * y, axis=0, keepdims=True)

  @pl.when(i == 1)
  def _phase1():
    mu = s1[...] / N
    var = s2[...] / N - mu * mu
    y = y_sc[pl.ds(j * BR, BR), :]
    yn = (y - mu) * lax.rsqrt(var + EPS_BN) * g1_ref[...] + be1_ref[...]
    yn = jnp.maximum(yn, 0.0)
    z = jnp.dot(yn, W2_ref[...], preferred_element_type=jnp.float32)
    z = z + b2_ref[...]
    mu2 = jnp.mean(z, axis=1, keepdims=True)
    var2 = jnp.mean(z * z, axis=1, keepdims=True) - mu2 * mu2
    z = (z - mu2) * lax.rsqrt(var2 + EPS_LN) * g2_ref[...] + be2_ref[...]
    o = z + h_ref[...]
    out_ref[...] = o

    if pool is not None:
      mean_ref, max_ref, add_ref, psum, pmax = pool

      @pl.when(j == 0)
      def _():
        psum[...] = jnp.zeros((1, D), jnp.float32)
        pmax[...] = jnp.full((1, D), -jnp.inf, jnp.float32)

      psum[...] += jnp.sum(o, axis=0, keepdims=True)
      pmax[...] = jnp.maximum(pmax[...], jnp.max(o, axis=0, keepdims=True))

      @pl.when(j == NB - 1)
      def _():
        s = psum[...]
        add_ref[...] = s
        mean_ref[...] = s / N
        max_ref[...] = pmax[...]


def _tc_layer(h, parts, W1, b1, g1, be1, W2, b2, g2, be2, last):
  row_spec = pl.BlockSpec((BR, D), lambda i, j: (j, 0))
  p_spec = pl.BlockSpec((NC, BR, D), lambda i, j: (0, (1 - i) * j, 0))
  w_spec = pl.BlockSpec((D, D), lambda i, j: (0, 0))
  v_spec = pl.BlockSpec((1, D), lambda i, j: (0, 0))
  out_row_spec = pl.BlockSpec((BR, D), lambda i, j: (i * j, 0))

  in_specs = [row_spec, p_spec, w_spec, v_spec, v_spec, v_spec,
              w_spec, v_spec, v_spec, v_spec]
  scratch = [
      pltpu.VMEM((N, D), jnp.float32),
      pltpu.VMEM((1, D), jnp.float32),
      pltpu.VMEM((1, D), jnp.float32),
  ]
  if last:
    out_shape = [
        jax.ShapeDtypeStruct((N, D), jnp.float32),
        jax.ShapeDtypeStruct((1, D), jnp.float32),
        jax.ShapeDtypeStruct((1, D), jnp.float32),
        jax.ShapeDtypeStruct((1, D), jnp.float32),
    ]
    out_specs = [out_row_spec, v_spec, v_spec, v_spec]
    body = _tc_body_last
    scratch += [pltpu.VMEM((1, D), jnp.float32),
                pltpu.VMEM((1, D), jnp.float32)]
  else:
    out_shape = jax.ShapeDtypeStruct((N, D), jnp.float32)
    out_specs = out_row_spec
    body = _tc_body_mid

  return pl.pallas_call(
      body,
      grid=(2, NB),
      in_specs=in_specs,
      out_specs=out_specs,
      out_shape=out_shape,
      scratch_shapes=scratch,
  )(h, parts, W1, b1, g1, be1, W2, b2, g2, be2)


def kernel(x, edge_index,
           W1_0, b1_0, bng_0, bnb_0, W2_0, b2_0, lng_0, lnb_0,
           W1_1, b1_1, bng_1, bnb_1, W2_1, b2_1, lng_1, lnb_1,
           W1_2, b1_2, bng_2, bnb_2, W2_2, b2_2, lng_2, lnb_2):
  ei = edge_index.reshape(2, NW, EPT)
  idx = jnp.stack(
      [ei[0].reshape(NW, NCHUNK, CH), ei[1].reshape(NW, NCHUNK, CH)],
      axis=2).reshape(NC, NS, NCHUNK, 2, CH)
  zeros = jnp.zeros((RPT, D), jnp.float32)
  params = [
      (W1_0, b1_0, bng_0, bnb_0, W2_0, b2_0, lng_0, lnb_0),
      (W1_1, b1_1, bng_1, bnb_1, W2_1, b2_1, lng_1, lnb_1),
      (W1_2, b1_2, bng_2, bnb_2, W2_2, b2_2, lng_2, lnb_2),
  ]
  h = x
  outs = None
  for i in range(3):
    W1, b1, g1, be1, W2, b2, g2, be2 = params[i]
    parts = _seg_sum(h, idx, zeros)
    res = _tc_layer(h, parts,
                    W1, b1.reshape(1, D), g1.reshape(1, D), be1.reshape(1, D),
                    W2, b2.reshape(1, D), g2.reshape(1, D), be2.reshape(1, D),
                    last=(i == 2))
    if i == 2:
      h, mean_p, max_p, add_p = res
      outs = (mean_p, max_p, add_p, h)
    else:
      h = res
  return outs


# speedup vs baseline: 2.6789x; 1.2409x over previous
import functools

import jax
import jax.numpy as jnp
from jax import lax
from jax.experimental import pallas as pl
from jax.experimental.pallas import tpu as pltpu
from jax.experimental.pallas import tpu_sc as plsc

N = 10000
E = 320000
D = 128
EPS_BN = 1e-5
EPS_LN = 1e-5

NC = 2
NS = 16
NW = NC * NS
EPT = E // NW
CH = 100
NCHUNK = EPT // CH
CPI = 10
NIT = NCHUNK // CPI
ACC_N = 10112
RPT = ACC_N // NS


@functools.cache
def _make_seg_sum():
  mesh = plsc.VectorSubcoreMesh(core_axis_name="c", subcore_axis_name="s",
                                num_cores=NC, num_subcores=NS)

  @functools.partial(
      pl.kernel,
      out_type=jax.ShapeDtypeStruct((NC, ACC_N, D), jnp.float32),
      mesh=mesh,
      scratch_types=[
          pltpu.VMEM_SHARED((ACC_N, D), jnp.float32),
          pltpu.VMEM((CPI, 2, CH), jnp.int32),
          pltpu.VMEM((CPI, 2, CH), jnp.int32),
          pltpu.VMEM((CH, D), jnp.float32),
          pltpu.VMEM((CH, D), jnp.float32),
          pltpu.SemaphoreType.DMA,
          pltpu.SemaphoreType.DMA,
          pltpu.SemaphoreType.DMA,
          pltpu.SemaphoreType.DMA,
      ],
  )
  def seg(h_hbm, idx_hbm, z_hbm, out_hbm,
          acc, ib0, ib1, r0, r1, gs0, gs1, is0, is1):
    cid = lax.axis_index("c")
    sid = lax.axis_index("s")
    pltpu.sync_copy(z_hbm, acc.at[pl.ds(sid * RPT, RPT)])
    plsc.subcore_barrier()

    ibufs = [ib0, ib1]
    isems = [is0, is1]
    rbufs = [r0, r1]
    gsems = [gs0, gs1]

    def iload(t, p):
      pltpu.async_copy(idx_hbm.at[cid, sid, pl.ds(t * CPI, CPI)],
                       ibufs[p], isems[p])

    def iwait(p):
      pltpu.make_async_copy(idx_hbm.at[cid, sid, pl.ds(0, CPI)],
                            ibufs[p], isems[p]).wait()

    iload(0, 0)

    def one_iter(t, p):
      iwait(p)
      iload(jnp.minimum(t + 1, NIT - 1), 1 - p)
      ib = ibufs[p]
      d = pltpu.async_copy(h_hbm.at[ib.at[0, 0]], rbufs[0], gsems[0])
      for k in range(CPI):
        d.wait()
        if k + 1 < CPI:
          d = pltpu.async_copy(h_hbm.at[ib.at[k + 1, 0]],
                               rbufs[(k + 1) % 2], gsems[(k + 1) % 2])
        pltpu.sync_copy(rbufs[k % 2], acc.at[ib.at[k, 1]], add=True)

    def body(s, carry):
      one_iter(2 * s, 0)
      one_iter(2 * s + 1, 1)
      return carry

    lax.fori_loop(0, NIT // 2, body, 0)
    iwait(0)
    plsc.subcore_barrier()
    pltpu.sync_copy(acc.at[pl.ds(sid * RPT, RPT)],
                    out_hbm.at[cid, pl.ds(sid * RPT, RPT)])

  return seg


def _seg_sum(h, idx, zeros):
  return _make_seg_sum()(h, idx, zeros)


BR = 1000
NB = N // BR


def _tc_body_last(h_ref, p_ref, W1_ref, b1_ref, g1_ref, be1_ref,
                  W2_ref, b2_ref, g2_ref, be2_ref,
                  out_ref, mean_ref, max_ref, add_ref,
                  y_sc, s1, s2, psum, pmax):
  _tc_common(h_ref, p_ref, W1_ref, b1_ref, g1_ref, be1_ref,
             W2_ref, b2_ref, g2_ref, be2_ref, out_ref,
             y_sc, s1, s2,
             pool=(mean_ref, max_ref, add_ref, psum, pmax))


def _tc_body_mid(h_ref, p_ref, W1_ref, b1_ref, g1_ref, be1_ref,
                 W2_ref, b2_ref, g2_ref, be2_ref,
                 out_ref, y_sc, s1, s2):
  _tc_common(h_ref, p_ref, W1_ref, b1_ref, g1_ref, be1_ref,
             W2_ref, b2_ref, g2_ref, be2_ref, out_ref,
             y_sc, s1, s2, pool=None)


def _tc_common(h_ref, p_ref, W1_ref, b1_ref, g1_ref, be1_ref,
               W2_ref, b2_ref, g2_ref, be2_ref, out_ref,
               y_sc, s1, s2, pool):
  i = pl.program_id(0)
  j = pl.program_id(1)

  @pl.when(i == 0)
  def _phase0():
    m = h_ref[...] + p_ref[0] + p_ref[1]
    y = jnp.dot(m, W1_ref[...], preferred_element_type=jnp.float32)
    y = y + b1_ref[...]
    y_sc[pl.ds(j * BR, BR), :] = y

    @pl.when(j == 0)
    def _():
      s1[...] = jnp.zeros((1, D), jnp.float32)
      s2[...] = jnp.zeros((1, D), jnp.float32)

    s1[...] += jnp.sum(y, axis=0, keepdims=True)
    s2[...] += jnp.sum(y * y, axis=0, keepdims=True)

  @pl.when(i == 1)
  def _phase1():
    mu = s1[...] / N
    var = s2[...] / N - mu * mu
    y = y_sc[pl.ds(j * BR, BR), :]
    yn = (y - mu) * lax.rsqrt(var + EPS_BN) * g1_ref[...] + be1_ref[...]
    yn = jnp.maximum(yn, 0.0)
    z = jnp.dot(yn, W2_ref[...], preferred_element_type=jnp.float32)
    z = z + b2_ref[...]
    mu2 = jnp.mean(z, axis=1, keepdims=True)
    var2 = jnp.mean(z * z, axis=1, keepdims=True) - mu2 * mu2
    z = (z - mu2) * lax.rsqrt(var2 + EPS_LN) * g2_ref[...] + be2_ref[...]
    o = z + h_ref[...]
    out_ref[...] = o

    if pool is not None:
      mean_ref, max_ref, add_ref, psum, pmax = pool

      @pl.when(j == 0)
      def _():
        psum[...] = jnp.zeros((1, D), jnp.float32)
        pmax[...] = jnp.full((1, D), -jnp.inf, jnp.float32)

      psum[...] += jnp.sum(o, axis=0, keepdims=True)
      pmax[...] = jnp.maximum(pmax[...], jnp.max(o, axis=0, keepdims=True))

      @pl.when(j == NB - 1)
      def _():
        s = psum[...]
        add_ref[...] = s
        mean_ref[...] = s / N
        max_ref[...] = pmax[...]


def _tc_layer(h, parts, W1, b1, g1, be1, W2, b2, g2, be2, last):
  row_spec = pl.BlockSpec((BR, D), lambda i, j: (j, 0))
  p_spec = pl.BlockSpec((NC, BR, D), lambda i, j: (0, (1 - i) * j, 0))
  w_spec = pl.BlockSpec((D, D), lambda i, j: (0, 0))
  v_spec = pl.BlockSpec((1, D), lambda i, j: (0, 0))
  out_row_spec = pl.BlockSpec((BR, D), lambda i, j: (i * j, 0))

  in_specs = [row_spec, p_spec, w_spec, v_spec, v_spec, v_spec,
              w_spec, v_spec, v_spec, v_spec]
  scratch = [
      pltpu.VMEM((N, D), jnp.float32),
      pltpu.VMEM((1, D), jnp.float32),
      pltpu.VMEM((1, D), jnp.float32),
  ]
  if last:
    out_shape = [
        jax.ShapeDtypeStruct((N, D), jnp.float32),
        jax.ShapeDtypeStruct((1, D), jnp.float32),
        jax.ShapeDtypeStruct((1, D), jnp.float32),
        jax.ShapeDtypeStruct((1, D), jnp.float32),
    ]
    out_specs = [out_row_spec, v_spec, v_spec, v_spec]
    body = _tc_body_last
    scratch += [pltpu.VMEM((1, D), jnp.float32),
                pltpu.VMEM((1, D), jnp.float32)]
  else:
    out_shape = jax.ShapeDtypeStruct((N, D), jnp.float32)
    out_specs = out_row_spec
    body = _tc_body_mid

  return pl.pallas_call(
      body,
      grid=(2, NB),
      in_specs=in_specs,
      out_specs=out_specs,
      out_shape=out_shape,
      scratch_shapes=scratch,
  )(h, parts, W1, b1, g1, be1, W2, b2, g2, be2)


def kernel(x, edge_index,
           W1_0, b1_0, bng_0, bnb_0, W2_0, b2_0, lng_0, lnb_0,
           W1_1, b1_1, bng_1, bnb_1, W2_1, b2_1, lng_1, lnb_1,
           W1_2, b1_2, bng_2, bnb_2, W2_2, b2_2, lng_2, lnb_2):
  ei = edge_index.reshape(2, NW, EPT)
  idx = jnp.stack(
      [ei[0].reshape(NW, NCHUNK, CH), ei[1].reshape(NW, NCHUNK, CH)],
      axis=2).reshape(NC, NS, NCHUNK, 2, CH)
  zeros = jnp.zeros((RPT, D), jnp.float32)
  params = [
      (W1_0, b1_0, bng_0, bnb_0, W2_0, b2_0, lng_0, lnb_0),
      (W1_1, b1_1, bng_1, bnb_1, W2_1, b2_1, lng_1, lnb_1),
      (W1_2, b1_2, bng_2, bnb_2, W2_2, b2_2, lng_2, lnb_2),
  ]
  h = x
  outs = None
  for i in range(3):
    W1, b1, g1, be1, W2, b2, g2, be2 = params[i]
    parts = _seg_sum(h, idx, zeros)
    res = _tc_layer(h, parts,
                    W1, b1.reshape(1, D), g1.reshape(1, D), be1.reshape(1, D),
                    W2, b2.reshape(1, D), g2.reshape(1, D), be2.reshape(1, D),
                    last=(i == 2))
    if i == 2:
      h, mean_p, max_p, add_p = res
      outs = (mean_p, max_p, add_p, h)
    else:
      h = res
  return outs


# speedup vs baseline: 2.7097x; 1.0115x over previous
import functools

import jax
import jax.numpy as jnp
from jax import lax
from jax.experimental import pallas as pl
from jax.experimental.pallas import tpu as pltpu
from jax.experimental.pallas import tpu_sc as plsc

N = 10000
E = 320000
D = 128
EPS_BN = 1e-5
EPS_LN = 1e-5

NC = 2
NS = 16
NW = NC * NS
EPT = E // NW
CH = 100
NCHUNK = EPT // CH
CPI = 20
NIT = NCHUNK // CPI
ACC_N = 10112
RPT = ACC_N // NS


@functools.cache
def _make_seg_sum():
  mesh = plsc.VectorSubcoreMesh(core_axis_name="c", subcore_axis_name="s",
                                num_cores=NC, num_subcores=NS)

  @functools.partial(
      pl.kernel,
      out_type=jax.ShapeDtypeStruct((NC, ACC_N, D), jnp.float32),
      mesh=mesh,
      scratch_types=[
          pltpu.VMEM_SHARED((ACC_N, D), jnp.float32),
          pltpu.VMEM((CPI, 2, CH), jnp.int32),
          pltpu.VMEM((CPI, 2, CH), jnp.int32),
          pltpu.VMEM((CH, D), jnp.float32),
          pltpu.VMEM((CH, D), jnp.float32),
          pltpu.SemaphoreType.DMA,
          pltpu.SemaphoreType.DMA,
          pltpu.SemaphoreType.DMA,
          pltpu.SemaphoreType.DMA,
      ],
  )
  def seg(h_hbm, idx_hbm, z_hbm, out_hbm,
          acc, ib0, ib1, r0, r1, gs0, gs1, is0, is1):
    cid = lax.axis_index("c")
    sid = lax.axis_index("s")
    pltpu.sync_copy(z_hbm, acc.at[pl.ds(sid * RPT, RPT)])
    plsc.subcore_barrier()

    ibufs = [ib0, ib1]
    isems = [is0, is1]
    rbufs = [r0, r1]
    gsems = [gs0, gs1]

    def iload(t, p):
      pltpu.async_copy(idx_hbm.at[cid, sid, pl.ds(t * CPI, CPI)],
                       ibufs[p], isems[p])

    def iwait(p):
      pltpu.make_async_copy(idx_hbm.at[cid, sid, pl.ds(0, CPI)],
                            ibufs[p], isems[p]).wait()

    iload(0, 0)

    def one_iter(t, p):
      iwait(p)
      iload(jnp.minimum(t + 1, NIT - 1), 1 - p)
      ib = ibufs[p]
      d = pltpu.async_copy(h_hbm.at[ib.at[0, 0]], rbufs[0], gsems[0])
      for k in range(CPI):
        d.wait()
        if k + 1 < CPI:
          d = pltpu.async_copy(h_hbm.at[ib.at[k + 1, 0]],
                               rbufs[(k + 1) % 2], gsems[(k + 1) % 2])
        pltpu.sync_copy(rbufs[k % 2], acc.at[ib.at[k, 1]], add=True)

    def body(s, carry):
      one_iter(2 * s, 0)
      one_iter(2 * s + 1, 1)
      return carry

    lax.fori_loop(0, NIT // 2, body, 0)
    if NIT % 2 == 1:
      one_iter(NIT - 1, 0)
    iwait(1 - ((NIT - 1) % 2))
    plsc.subcore_barrier()
    pltpu.sync_copy(acc.at[pl.ds(sid * RPT, RPT)],
                    out_hbm.at[cid, pl.ds(sid * RPT, RPT)])

  return seg


def _seg_sum(h, idx, zeros):
  return _make_seg_sum()(h, idx, zeros)


BR = 1000
NB = N // BR


def _tc_body_last(h_ref, p_ref, W1_ref, b1_ref, g1_ref, be1_ref,
                  W2_ref, b2_ref, g2_ref, be2_ref,
                  out_ref, mean_ref, max_ref, add_ref,
                  y_sc, s1, s2, psum, pmax):
  _tc_common(h_ref, p_ref, W1_ref, b1_ref, g1_ref, be1_ref,
             W2_ref, b2_ref, g2_ref, be2_ref, out_ref,
             y_sc, s1, s2,
             pool=(mean_ref, max_ref, add_ref, psum, pmax))


def _tc_body_mid(h_ref, p_ref, W1_ref, b1_ref, g1_ref, be1_ref,
                 W2_ref, b2_ref, g2_ref, be2_ref,
                 out_ref, y_sc, s1, s2):
  _tc_common(h_ref, p_ref, W1_ref, b1_ref, g1_ref, be1_ref,
             W2_ref, b2_ref, g2_ref, be2_ref, out_ref,
             y_sc, s1, s2, pool=None)


def _tc_common(h_ref, p_ref, W1_ref, b1_ref, g1_ref, be1_ref,
               W2_ref, b2_ref, g2_ref, be2_ref, out_ref,
               y_sc, s1, s2, pool):
  i = pl.program_id(0)
  j = pl.program_id(1)

  @pl.when(i == 0)
  def _phase0():
    m = h_ref[...] + p_ref[0] + p_ref[1]
    y = jnp.dot(m, W1_ref[...], preferred_element_type=jnp.float32)
    y = y + b1_ref[...]
    y_sc[pl.ds(j * BR, BR), :] = y

    @pl.when(j == 0)
    def _():
      s1[...] = jnp.zeros((1, D), jnp.float32)
      s2[...] = jnp.zeros((1, D), jnp.float32)

    s1[...] += jnp.sum(y, axis=0, keepdims=True)
    s2[...] += jnp.sum(y * y, axis=0, keepdims=True)

  @pl.when(i == 1)
  def _phase1():
    mu = s1[...] / N
    var = s2[...] / N - mu * mu
    y = y_sc[pl.ds(j * BR, BR), :]
    yn = (y - mu) * lax.rsqrt(var + EPS_BN) * g1_ref[...] + be1_ref[...]
    yn = jnp.maximum(yn, 0.0)
    z = jnp.dot(yn, W2_ref[...], preferred_element_type=jnp.float32)
    z = z + b2_ref[...]
    mu2 = jnp.mean(z, axis=1, keepdims=True)
    var2 = jnp.mean(z * z, axis=1, keepdims=True) - mu2 * mu2
    z = (z - mu2) * lax.rsqrt(var2 + EPS_LN) * g2_ref[...] + be2_ref[...]
    o = z + h_ref[...]
    out_ref[...] = o

    if pool is not None:
      mean_ref, max_ref, add_ref, psum, pmax = pool

      @pl.when(j == 0)
      def _():
        psum[...] = jnp.zeros((1, D), jnp.float32)
        pmax[...] = jnp.full((1, D), -jnp.inf, jnp.float32)

      psum[...] += jnp.sum(o, axis=0, keepdims=True)
      pmax[...] = jnp.maximum(pmax[...], jnp.max(o, axis=0, keepdims=True))

      @pl.when(j == NB - 1)
      def _():
        s = psum[...]
        add_ref[...] = s
        mean_ref[...] = s / N
        max_ref[...] = pmax[...]


def _tc_layer(h, parts, W1, b1, g1, be1, W2, b2, g2, be2, last):
  row_spec = pl.BlockSpec((BR, D), lambda i, j: (j, 0))
  p_spec = pl.BlockSpec((NC, BR, D), lambda i, j: (0, (1 - i) * j, 0))
  w_spec = pl.BlockSpec((D, D), lambda i, j: (0, 0))
  v_spec = pl.BlockSpec((1, D), lambda i, j: (0, 0))
  out_row_spec = pl.BlockSpec((BR, D), lambda i, j: (i * j, 0))

  in_specs = [row_spec, p_spec, w_spec, v_spec, v_spec, v_spec,
              w_spec, v_spec, v_spec, v_spec]
  scratch = [
      pltpu.VMEM((N, D), jnp.float32),
      pltpu.VMEM((1, D), jnp.float32),
      pltpu.VMEM((1, D), jnp.float32),
  ]
  if last:
    out_shape = [
        jax.ShapeDtypeStruct((N, D), jnp.float32),
        jax.ShapeDtypeStruct((1, D), jnp.float32),
        jax.ShapeDtypeStruct((1, D), jnp.float32),
        jax.ShapeDtypeStruct((1, D), jnp.float32),
    ]
    out_specs = [out_row_spec, v_spec, v_spec, v_spec]
    body = _tc_body_last
    scratch += [pltpu.VMEM((1, D), jnp.float32),
                pltpu.VMEM((1, D), jnp.float32)]
  else:
    out_shape = jax.ShapeDtypeStruct((N, D), jnp.float32)
    out_specs = out_row_spec
    body = _tc_body_mid

  return pl.pallas_call(
      body,
      grid=(2, NB),
      in_specs=in_specs,
      out_specs=out_specs,
      out_shape=out_shape,
      scratch_shapes=scratch,
  )(h, parts, W1, b1, g1, be1, W2, b2, g2, be2)


def kernel(x, edge_index,
           W1_0, b1_0, bng_0, bnb_0, W2_0, b2_0, lng_0, lnb_0,
           W1_1, b1_1, bng_1, bnb_1, W2_1, b2_1, lng_1, lnb_1,
           W1_2, b1_2, bng_2, bnb_2, W2_2, b2_2, lng_2, lnb_2):
  ei = edge_index.reshape(2, NW, EPT)
  idx = jnp.stack(
      [ei[0].reshape(NW, NCHUNK, CH), ei[1].reshape(NW, NCHUNK, CH)],
      axis=2).reshape(NC, NS, NCHUNK, 2, CH)
  zeros = jnp.zeros((RPT, D), jnp.float32)
  params = [
      (W1_0, b1_0, bng_0, bnb_0, W2_0, b2_0, lng_0, lnb_0),
      (W1_1, b1_1, bng_1, bnb_1, W2_1, b2_1, lng_1, lnb_1),
      (W1_2, b1_2, bng_2, bnb_2, W2_2, b2_2, lng_2, lnb_2),
  ]
  h = x
  outs = None
  for i in range(3):
    W1, b1, g1, be1, W2, b2, g2, be2 = params[i]
    parts = _seg_sum(h, idx, zeros)
    res = _tc_layer(h, parts,
                    W1, b1.reshape(1, D), g1.reshape(1, D), be1.reshape(1, D),
                    W2, b2.reshape(1, D), g2.reshape(1, D), be2.reshape(1, D),
                    last=(i == 2))
    if i == 2:
      h, mean_p, max_p, add_p = res
      outs = (mean_p, max_p, add_p, h)
    else:
      h = res
  return outs


# speedup vs baseline: 2.8377x; 1.0472x over previous
import functools

import jax
import jax.numpy as jnp
from jax import lax
from jax.experimental import pallas as pl
from jax.experimental.pallas import tpu as pltpu
from jax.experimental.pallas import tpu_sc as plsc

N = 10000
E = 320000
D = 128
EPS_BN = 1e-5
EPS_LN = 1e-5

NC = 2
NS = 16
NW = NC * NS
EPT = E // NW
CH = 100
NCHUNK = EPT // CH
CPI = 20
NIT = NCHUNK // CPI
ACC_N = 10112
RPT = ACC_N // NS


@functools.cache
def _make_seg_sum():
  mesh = plsc.VectorSubcoreMesh(core_axis_name="c", subcore_axis_name="s",
                                num_cores=NC, num_subcores=NS)

  @functools.partial(
      pl.kernel,
      out_type=jax.ShapeDtypeStruct((NC, ACC_N, D), jnp.float32),
      mesh=mesh,
      scratch_types=[
          pltpu.VMEM_SHARED((ACC_N, D), jnp.float32),
          pltpu.VMEM((CPI, 2, CH), jnp.int32),
          pltpu.VMEM((CPI, 2, CH), jnp.int32),
          pltpu.VMEM((CH, D), jnp.float32),
          pltpu.VMEM((CH, D), jnp.float32),
          pltpu.SemaphoreType.DMA,
          pltpu.SemaphoreType.DMA,
          pltpu.SemaphoreType.DMA,
          pltpu.SemaphoreType.DMA,
      ],
  )
  def seg(h_hbm, idx_hbm, z_hbm, out_hbm,
          acc, ib0, ib1, r0, r1, gs0, gs1, is0, is1):
    cid = lax.axis_index("c")
    sid = lax.axis_index("s")
    pltpu.sync_copy(z_hbm, acc.at[pl.ds(sid * RPT, RPT)])
    plsc.subcore_barrier()

    ibufs = [ib0, ib1]
    isems = [is0, is1]
    rbufs = [r0, r1]
    gsems = [gs0, gs1]

    def iload(t, p):
      pltpu.async_copy(idx_hbm.at[cid, sid, pl.ds(t * CPI, CPI)],
                       ibufs[p], isems[p])

    def iwait(p):
      pltpu.make_async_copy(idx_hbm.at[cid, sid, pl.ds(0, CPI)],
                            ibufs[p], isems[p]).wait()

    iload(0, 0)

    def one_iter(t, p):
      iwait(p)
      iload(jnp.minimum(t + 1, NIT - 1), 1 - p)
      ib = ibufs[p]
      d = pltpu.async_copy(h_hbm.at[ib.at[0, 0]], rbufs[0], gsems[0])
      for k in range(CPI):
        d.wait()
        if k + 1 < CPI:
          d = pltpu.async_copy(h_hbm.at[ib.at[k + 1, 0]],
                               rbufs[(k + 1) % 2], gsems[(k + 1) % 2])
        pltpu.sync_copy(rbufs[k % 2], acc.at[ib.at[k, 1]], add=True)

    def body(s, carry):
      one_iter(2 * s, 0)
      one_iter(2 * s + 1, 1)
      return carry

    lax.fori_loop(0, NIT // 2, body, 0)
    if NIT % 2 == 1:
      one_iter(NIT - 1, 0)
    iwait(1 - ((NIT - 1) % 2))
    plsc.subcore_barrier()
    pltpu.sync_copy(acc.at[pl.ds(sid * RPT, RPT)],
                    out_hbm.at[cid, pl.ds(sid * RPT, RPT)])

  return seg


def _seg_sum(h, idx, zeros):
  return _make_seg_sum()(h, idx, zeros)


BR = 1000
NB = N // BR


def _tc_compute(h_ref, p_ref, W1_ref, b1_ref, g1_ref, be1_ref,
                W2_ref, b2_ref, g2_ref, be2_ref):
  m = h_ref[...] + p_ref[0, :N, :] + p_ref[1, :N, :]
  y = jnp.dot(m, W1_ref[...], preferred_element_type=jnp.float32)
  y = y + b1_ref[...]
  mu = jnp.mean(y, axis=0, keepdims=True)
  var = jnp.mean(y * y, axis=0, keepdims=True) - mu * mu
  yn = (y - mu) * lax.rsqrt(var + EPS_BN) * g1_ref[...] + be1_ref[...]
  yn = jnp.maximum(yn, 0.0)
  z = jnp.dot(yn, W2_ref[...], preferred_element_type=jnp.float32)
  z = z + b2_ref[...]
  mu2 = jnp.mean(z, axis=1, keepdims=True)
  var2 = jnp.mean(z * z, axis=1, keepdims=True) - mu2 * mu2
  z = (z - mu2) * lax.rsqrt(var2 + EPS_LN) * g2_ref[...] + be2_ref[...]
  return z + h_ref[...]


def _tc_body_mid(h_ref, p_ref, W1_ref, b1_ref, g1_ref, be1_ref,
                 W2_ref, b2_ref, g2_ref, be2_ref, out_ref):
  out_ref[...] = _tc_compute(h_ref, p_ref, W1_ref, b1_ref, g1_ref, be1_ref,
                             W2_ref, b2_ref, g2_ref, be2_ref)


def _tc_body_last(h_ref, p_ref, W1_ref, b1_ref, g1_ref, be1_ref,
                  W2_ref, b2_ref, g2_ref, be2_ref,
                  out_ref, mean_ref, max_ref, add_ref):
  o = _tc_compute(h_ref, p_ref, W1_ref, b1_ref, g1_ref, be1_ref,
                  W2_ref, b2_ref, g2_ref, be2_ref)
  out_ref[...] = o
  s = jnp.sum(o, axis=0, keepdims=True)
  add_ref[...] = s
  mean_ref[...] = s / N
  max_ref[...] = jnp.max(o, axis=0, keepdims=True)


def _tc_layer(h, parts, W1, b1, g1, be1, W2, b2, g2, be2, last):
  if last:
    out_shape = [
        jax.ShapeDtypeStruct((N, D), jnp.float32),
        jax.ShapeDtypeStruct((1, D), jnp.float32),
        jax.ShapeDtypeStruct((1, D), jnp.float32),
        jax.ShapeDtypeStruct((1, D), jnp.float32),
    ]
    body = _tc_body_last
  else:
    out_shape = jax.ShapeDtypeStruct((N, D), jnp.float32)
    body = _tc_body_mid

  return pl.pallas_call(body, out_shape=out_shape)(
      h, parts, W1, b1, g1, be1, W2, b2, g2, be2)


def kernel(x, edge_index,
           W1_0, b1_0, bng_0, bnb_0, W2_0, b2_0, lng_0, lnb_0,
           W1_1, b1_1, bng_1, bnb_1, W2_1, b2_1, lng_1, lnb_1,
           W1_2, b1_2, bng_2, bnb_2, W2_2, b2_2, lng_2, lnb_2):
  ei = edge_index.reshape(2, NW, EPT)
  idx = jnp.stack(
      [ei[0].reshape(NW, NCHUNK, CH), ei[1].reshape(NW, NCHUNK, CH)],
      axis=2).reshape(NC, NS, NCHUNK, 2, CH)
  zeros = jnp.zeros((RPT, D), jnp.float32)
  params = [
      (W1_0, b1_0, bng_0, bnb_0, W2_0, b2_0, lng_0, lnb_0),
      (W1_1, b1_1, bng_1, bnb_1, W2_1, b2_1, lng_1, lnb_1),
      (W1_2, b1_2, bng_2, bnb_2, W2_2, b2_2, lng_2, lnb_2),
  ]
  h = x
  outs = None
  for i in range(3):
    W1, b1, g1, be1, W2, b2, g2, be2 = params[i]
    parts = _seg_sum(h, idx, zeros)
    res = _tc_layer(h, parts,
                    W1, b1.reshape(1, D), g1.reshape(1, D), be1.reshape(1, D),
                    W2, b2.reshape(1, D), g2.reshape(1, D), be2.reshape(1, D),
                    last=(i == 2))
    if i == 2:
      h, mean_p, max_p, add_p = res
      outs = (mean_p, max_p, add_p, h)
    else:
      h = res
  return outs


# speedup vs baseline: 3.0463x; 1.0735x over previous
import functools

import jax
import jax.numpy as jnp
from jax import lax
from jax.experimental import pallas as pl
from jax.experimental.pallas import tpu as pltpu
from jax.experimental.pallas import tpu_sc as plsc

N = 10000
E = 320000
D = 128
EPS_BN = 1e-5
EPS_LN = 1e-5

NC = 2
NS = 16
NW = NC * NS
EPT = E // NW
CH = 125
NCHUNK = EPT // CH
CPI = 20
NIT = NCHUNK // CPI
ACC_N = 10112
RPT = ACC_N // NS


@functools.cache
def _make_seg_sum():
  mesh = plsc.VectorSubcoreMesh(core_axis_name="c", subcore_axis_name="s",
                                num_cores=NC, num_subcores=NS)

  @functools.partial(
      pl.kernel,
      out_type=jax.ShapeDtypeStruct((NC, ACC_N, D), jnp.float32),
      mesh=mesh,
      scratch_types=[
          pltpu.VMEM_SHARED((ACC_N, D), jnp.float32),
          pltpu.VMEM((CPI, 2, CH), jnp.int32),
          pltpu.VMEM((CPI, 2, CH), jnp.int32),
          pltpu.VMEM((CH, D), jnp.float32),
          pltpu.VMEM((CH, D), jnp.float32),
          pltpu.SemaphoreType.DMA,
          pltpu.SemaphoreType.DMA,
          pltpu.SemaphoreType.DMA,
          pltpu.SemaphoreType.DMA,
      ],
  )
  def seg(h_hbm, idx_hbm, z_hbm, out_hbm,
          acc, ib0, ib1, r0, r1, gs0, gs1, is0, is1):
    cid = lax.axis_index("c")
    sid = lax.axis_index("s")
    pltpu.sync_copy(z_hbm, acc.at[pl.ds(sid * RPT, RPT)])
    plsc.subcore_barrier()

    ibufs = [ib0, ib1]
    isems = [is0, is1]
    rbufs = [r0, r1]
    gsems = [gs0, gs1]

    def iload(t, p):
      pltpu.async_copy(idx_hbm.at[cid, sid, pl.ds(t * CPI, CPI)],
                       ibufs[p], isems[p])

    def iwait(p):
      pltpu.make_async_copy(idx_hbm.at[cid, sid, pl.ds(0, CPI)],
                            ibufs[p], isems[p]).wait()

    iload(0, 0)

    def one_iter(t, p):
      iwait(p)
      iload(jnp.minimum(t + 1, NIT - 1), 1 - p)
      ib = ibufs[p]
      d = pltpu.async_copy(h_hbm.at[ib.at[0, 0]], rbufs[0], gsems[0])
      for k in range(CPI):
        d.wait()
        if k + 1 < CPI:
          d = pltpu.async_copy(h_hbm.at[ib.at[k + 1, 0]],
                               rbufs[(k + 1) % 2], gsems[(k + 1) % 2])
        pltpu.sync_copy(rbufs[k % 2], acc.at[ib.at[k, 1]], add=True)

    def body(s, carry):
      one_iter(2 * s, 0)
      one_iter(2 * s + 1, 1)
      return carry

    lax.fori_loop(0, NIT // 2, body, 0)
    if NIT % 2 == 1:
      one_iter(NIT - 1, 0)
    iwait(1 - ((NIT - 1) % 2))
    plsc.subcore_barrier()
    pltpu.sync_copy(acc.at[pl.ds(sid * RPT, RPT)],
                    out_hbm.at[cid, pl.ds(sid * RPT, RPT)])

  return seg


def _seg_sum(h, idx, zeros):
  return _make_seg_sum()(h, idx, zeros)


BR = 1000
NB = N // BR


def _tc_compute(h_ref, p_ref, W1_ref, b1_ref, g1_ref, be1_ref,
                W2_ref, b2_ref, g2_ref, be2_ref):
  m = h_ref[...] + p_ref[0, :N, :] + p_ref[1, :N, :]
  y = jnp.dot(m, W1_ref[...], preferred_element_type=jnp.float32)
  y = y + b1_ref[...]
  mu = jnp.mean(y, axis=0, keepdims=True)
  var = jnp.mean(y * y, axis=0, keepdims=True) - mu * mu
  yn = (y - mu) * lax.rsqrt(var + EPS_BN) * g1_ref[...] + be1_ref[...]
  yn = jnp.maximum(yn, 0.0)
  z = jnp.dot(yn, W2_ref[...], preferred_element_type=jnp.float32)
  z = z + b2_ref[...]
  mu2 = jnp.mean(z, axis=1, keepdims=True)
  var2 = jnp.mean(z * z, axis=1, keepdims=True) - mu2 * mu2
  z = (z - mu2) * lax.rsqrt(var2 + EPS_LN) * g2_ref[...] + be2_ref[...]
  return z + h_ref[...]


def _tc_body_mid(h_ref, p_ref, W1_ref, b1_ref, g1_ref, be1_ref,
                 W2_ref, b2_ref, g2_ref, be2_ref, out_ref):
  out_ref[...] = _tc_compute(h_ref, p_ref, W1_ref, b1_ref, g1_ref, be1_ref,
                             W2_ref, b2_ref, g2_ref, be2_ref)


def _tc_body_last(h_ref, p_ref, W1_ref, b1_ref, g1_ref, be1_ref,
                  W2_ref, b2_ref, g2_ref, be2_ref,
                  out_ref, mean_ref, max_ref, add_ref):
  o = _tc_compute(h_ref, p_ref, W1_ref, b1_ref, g1_ref, be1_ref,
                  W2_ref, b2_ref, g2_ref, be2_ref)
  out_ref[...] = o
  s = jnp.sum(o, axis=0, keepdims=True)
  add_ref[...] = s
  mean_ref[...] = s / N
  max_ref[...] = jnp.max(o, axis=0, keepdims=True)


def _tc_layer(h, parts, W1, b1, g1, be1, W2, b2, g2, be2, last):
  if last:
    out_shape = [
        jax.ShapeDtypeStruct((N, D), jnp.float32),
        jax.ShapeDtypeStruct((1, D), jnp.float32),
        jax.ShapeDtypeStruct((1, D), jnp.float32),
        jax.ShapeDtypeStruct((1, D), jnp.float32),
    ]
    body = _tc_body_last
  else:
    out_shape = jax.ShapeDtypeStruct((N, D), jnp.float32)
    body = _tc_body_mid

  return pl.pallas_call(body, out_shape=out_shape)(
      h, parts, W1, b1, g1, be1, W2, b2, g2, be2)


def kernel(x, edge_index,
           W1_0, b1_0, bng_0, bnb_0, W2_0, b2_0, lng_0, lnb_0,
           W1_1, b1_1, bng_1, bnb_1, W2_1, b2_1, lng_1, lnb_1,
           W1_2, b1_2, bng_2, bnb_2, W2_2, b2_2, lng_2, lnb_2):
  ei = edge_index.reshape(2, NW, EPT)
  idx = jnp.stack(
      [ei[0].reshape(NW, NCHUNK, CH), ei[1].reshape(NW, NCHUNK, CH)],
      axis=2).reshape(NC, NS, NCHUNK, 2, CH)
  zeros = jnp.zeros((RPT, D), jnp.float32)
  params = [
      (W1_0, b1_0, bng_0, bnb_0, W2_0, b2_0, lng_0, lnb_0),
      (W1_1, b1_1, bng_1, bnb_1, W2_1, b2_1, lng_1, lnb_1),
      (W1_2, b1_2, bng_2, bnb_2, W2_2, b2_2, lng_2, lnb_2),
  ]
  h = x
  outs = None
  for i in range(3):
    W1, b1, g1, be1, W2, b2, g2, be2 = params[i]
    parts = _seg_sum(h, idx, zeros)
    res = _tc_layer(h, parts,
                    W1, b1.reshape(1, D), g1.reshape(1, D), be1.reshape(1, D),
                    W2, b2.reshape(1, D), g2.reshape(1, D), be2.reshape(1, D),
                    last=(i == 2))
    if i == 2:
      h, mean_p, max_p, add_p = res
      outs = (mean_p, max_p, add_p, h)
    else:
      h = res
  return outs
